# Initial kernel scaffold; baseline (speedup 1.0000x reference)
import jax
import jax.numpy as jnp
from jax import lax
from jax.experimental import pallas as pl
from jax.experimental.pallas import tpu as pltpu
from jax.experimental.pallas import tpu_sc as plsc

B = 16
T = 2048
D = 128
MEL = 6144
L = 16
NC, NS = 2, 16
NW = NC * NS
ROWS_PER_TILE = B * MEL // NW
CHUNK = 128
NCHUNK = ROWS_PER_TILE // CHUNK
ZERO_ROW = B * T


def _lr_body(table_hbm, dur_hbm, out_hbm, mel_hbm, dur_v, idx_v, rows_v,
             mel_v, gsem):
    c = lax.axis_index("c")
    s = lax.axis_index("s")
    w = s * NC + c
    b = w // 2
    half = w % 2
    p0 = half * ROWS_PER_TILE
    out_base = w * ROWS_PER_TILE

    pltpu.sync_copy(dur_hbm.at[b], dur_v)

    zfill = jnp.full((L,), ZERO_ROW, dtype=jnp.int32)

    def init_step(i, _):
        idx_v[i // 8, pl.ds((i % 8) * L, L)] = zfill
        return 0

    lax.fori_loop(0, NCHUNK * (CHUNK // L), init_step, 0)

    iota = lax.iota(jnp.int32, L)

    def cum_step(j, carry):
        d = dur_v[pl.ds(j * L, L)]
        cinc = plsc.cumsum(d)
        cex = carry + cinc - d
        tok = b * T + j * L + iota
        for dd in range(3):
            pos = cex + dd
            m = (d > dd) & (pos >= p0) & (pos < p0 + ROWS_PER_TILE)
            lp = jnp.clip(pos - p0, 0, ROWS_PER_TILE - 1)
            r = lax.shift_right_logical(lp, 7)
            col = lax.bitwise_and(lp, 127)
            plsc.store_scatter(idx_v, [r, col], tok, mask=m)
        return carry + jnp.sum(d)

    total = lax.fori_loop(0, T // L, cum_step, jnp.int32(0))

    @pl.when(half == 0)
    def _():
        mel_v[...] = jnp.broadcast_to(total, (L,)).astype(jnp.int32)
        pltpu.sync_copy(mel_v.at[pl.ds(0, 8)], mel_hbm.at[b])

    def gather_step(j, _):
        pltpu.async_copy(table_hbm.at[idx_v.at[j]], rows_v, gsem).wait()
        pltpu.sync_copy(rows_v, out_hbm.at[pl.ds(out_base + j * CHUNK, CHUNK)])
        return 0

    lax.fori_loop(0, NCHUNK, gather_step, 0)


@jax.jit
def _length_regulate(table, dur):
    mesh = plsc.VectorSubcoreMesh(core_axis_name="c", subcore_axis_name="s")
    out, mel_pad = pl.kernel(
        _lr_body,
        mesh=mesh,
        out_type=[
            jax.ShapeDtypeStruct((B * MEL, D), jnp.float32),
            jax.ShapeDtypeStruct((B, 8), jnp.int32),
        ],
        scratch_types=[
            pltpu.VMEM((T,), jnp.int32),
            pltpu.VMEM((NCHUNK, CHUNK), jnp.int32),
            pltpu.VMEM((CHUNK, D), jnp.float32),
            pltpu.VMEM((L,), jnp.int32),
            pltpu.SemaphoreType.DMA,
        ],
    )(table, dur)
    return out, mel_pad


def kernel(x, x_pred_dur, max_dur):
    del max_dur
    table = jnp.concatenate(
        [x.reshape(B * T, D), jnp.zeros((8, D), x.dtype)], axis=0)
    dur = x_pred_dur.astype(jnp.int32)
    out, mel_pad = _length_regulate(table, dur)
    return out.reshape(B, MEL, D), mel_pad[:, 0]


# speedup vs baseline: 7.5961x; 7.5961x over previous
import jax
import jax.numpy as jnp
from jax import lax
from jax.experimental import pallas as pl
from jax.experimental.pallas import tpu as pltpu
from jax.experimental.pallas import tpu_sc as plsc

B = 16
T = 2048
D = 128
MEL = 6144
L = 16
NC, NS = 2, 16
NW = NC * NS
ROWS_PER_TILE = B * MEL // NW
CHUNK = 128
NCHUNK = ROWS_PER_TILE // CHUNK
ZERO_ROW = B * T


def _lr_body(table_hbm, dur_hbm, out_hbm, mel_hbm, dur_v, idx_v, rows_v,
             mel_v, gsem):
    c = lax.axis_index("c")
    s = lax.axis_index("s")
    w = s * NC + c
    b = w // 2
    half = w % 2
    p0 = half * ROWS_PER_TILE
    out_base = w * ROWS_PER_TILE

    pltpu.sync_copy(dur_hbm.at[b], dur_v)

    zfill = jnp.full((L,), ZERO_ROW, dtype=jnp.int32)

    def init_step(i, _):
        idx_v[i // 8, pl.ds((i % 8) * L, L)] = zfill
        return 0

    lax.fori_loop(0, NCHUNK * (CHUNK // L), init_step, 0)

    iota = lax.iota(jnp.int32, L)

    def cum_step(j, carry):
        d = dur_v[pl.ds(j * L, L)]
        cinc = plsc.cumsum(d)
        cex = carry + cinc - d
        tok = b * T + j * L + iota
        for dd in range(3):
            pos = cex + dd
            m = (d > dd) & (pos >= p0) & (pos < p0 + ROWS_PER_TILE)
            lp = jnp.clip(pos - p0, 0, ROWS_PER_TILE - 1)
            r = lax.shift_right_logical(lp, 7)
            col = lax.bitwise_and(lp, 127)
            plsc.store_scatter(idx_v, [r, col], tok, mask=m)
        return carry + jnp.sum(d)

    total = lax.fori_loop(0, T // L, cum_step, jnp.int32(0))

    @pl.when(half == 0)
    def _():
        tsplat = jnp.broadcast_to(total, (L,)).astype(jnp.int32)

        def fill_step(i, _):
            mel_v[pl.ds(i * L, L)] = tsplat
            return 0

        lax.fori_loop(0, 128 // L, fill_step, 0)
        pltpu.sync_copy(mel_v, mel_hbm.at[b])

    def gather_step(j, _):
        pltpu.async_copy(table_hbm.at[idx_v.at[j]], rows_v, gsem).wait()
        pltpu.sync_copy(rows_v, out_hbm.at[pl.ds(out_base + j * CHUNK, CHUNK)])
        return 0

    lax.fori_loop(0, NCHUNK, gather_step, 0)


@jax.jit
def _length_regulate(table, dur):
    mesh = plsc.VectorSubcoreMesh(core_axis_name="c", subcore_axis_name="s")
    out, mel_pad = pl.kernel(
        _lr_body,
        mesh=mesh,
        compiler_params=pltpu.CompilerParams(needs_layout_passes=False),
        out_type=[
            jax.ShapeDtypeStruct((B * MEL, D), jnp.float32),
            jax.ShapeDtypeStruct((B, 128), jnp.int32),
        ],
        scratch_types=[
            pltpu.VMEM((T,), jnp.int32),
            pltpu.VMEM((NCHUNK, CHUNK), jnp.int32),
            pltpu.VMEM((CHUNK, D), jnp.float32),
            pltpu.VMEM((128,), jnp.int32),
            pltpu.SemaphoreType.DMA,
        ],
    )(table, dur)
    return out, mel_pad


def kernel(x, x_pred_dur, max_dur):
    del max_dur
    table = jnp.concatenate(
        [x.reshape(B * T, D), jnp.zeros((8, D), x.dtype)], axis=0)
    dur = x_pred_dur.astype(jnp.int32)
    out, mel_pad = _length_regulate(table, dur)
    return out.reshape(B, MEL, D), mel_pad[:, 0]


# speedup vs baseline: 157.8237x; 20.7768x over previous
import jax
import jax.numpy as jnp
from jax import lax
from jax.experimental import pallas as pl
from jax.experimental.pallas import tpu as pltpu
from jax.experimental.pallas import tpu_sc as plsc

B = 16
T = 2048
D = 128
MEL = 6144
L = 16
NC, NS = 2, 16
NW = NC * NS
ROWS_PER_TILE = B * MEL // NW
CHUNK = 128
NCHUNK = ROWS_PER_TILE // CHUNK
NZERO = 1024
ZERO_ROW = B * T


def _lr_body(table_hbm, dur_hbm, out_hbm, mel_hbm, dur_v, idx_v, rows_v,
             mel_v, gsem):
    c = lax.axis_index("c")
    s = lax.axis_index("s")
    w = s * NC + c
    b = w // 2
    half = w % 2
    p0 = half * ROWS_PER_TILE
    out_base = w * ROWS_PER_TILE

    pltpu.sync_copy(dur_hbm.at[b], dur_v)

    iota0 = lax.iota(jnp.int32, L)

    def init_step(i, _):
        idx_v[i // 8, pl.ds((i % 8) * L, L)] = (
            ZERO_ROW + lax.bitwise_and(i * L + iota0, NZERO - 1))
        return 0

    lax.fori_loop(0, NCHUNK * (CHUNK // L), init_step, 0)

    iota = lax.iota(jnp.int32, L)

    def cum_step(j, carry):
        d = dur_v[pl.ds(j * L, L)]
        cinc = plsc.cumsum(d)
        cex = carry + cinc - d
        tok = b * T + j * L + iota
        for dd in range(3):
            pos = cex + dd
            m = (d > dd) & (pos >= p0) & (pos < p0 + ROWS_PER_TILE)
            lp = jnp.clip(pos - p0, 0, ROWS_PER_TILE - 1)
            r = lax.shift_right_logical(lp, 7)
            col = lax.bitwise_and(lp, 127)
            plsc.store_scatter(idx_v, [r, col], tok, mask=m)
        return carry + jnp.sum(d)

    total = lax.fori_loop(0, T // L, cum_step, jnp.int32(0))

    @pl.when(half == 0)
    def _():
        tsplat = jnp.broadcast_to(total, (L,)).astype(jnp.int32)

        def fill_step(i, _):
            mel_v[pl.ds(i * L, L)] = tsplat
            return 0

        lax.fori_loop(0, 128 // L, fill_step, 0)
        pltpu.sync_copy(mel_v, mel_hbm.at[b])

    def gather_step(j, _):
        pltpu.async_copy(table_hbm.at[idx_v.at[j]], rows_v, gsem).wait()
        pltpu.sync_copy(rows_v, out_hbm.at[pl.ds(out_base + j * CHUNK, CHUNK)])
        return 0

    lax.fori_loop(0, NCHUNK, gather_step, 0)


@jax.jit
def _length_regulate(table, dur):
    mesh = plsc.VectorSubcoreMesh(core_axis_name="c", subcore_axis_name="s")
    out, mel_pad = pl.kernel(
        _lr_body,
        mesh=mesh,
        compiler_params=pltpu.CompilerParams(needs_layout_passes=False),
        out_type=[
            jax.ShapeDtypeStruct((B * MEL, D), jnp.float32),
            jax.ShapeDtypeStruct((B, 128), jnp.int32),
        ],
        scratch_types=[
            pltpu.VMEM((T,), jnp.int32),
            pltpu.VMEM((NCHUNK, CHUNK), jnp.int32),
            pltpu.VMEM((CHUNK, D), jnp.float32),
            pltpu.VMEM((128,), jnp.int32),
            pltpu.SemaphoreType.DMA,
        ],
    )(table, dur)
    return out, mel_pad


def kernel(x, x_pred_dur, max_dur):
    del max_dur
    table = jnp.concatenate(
        [x.reshape(B * T, D), jnp.zeros((NZERO, D), x.dtype)], axis=0)
    dur = x_pred_dur.astype(jnp.int32)
    out, mel_pad = _length_regulate(table, dur)
    return out.reshape(B, MEL, D), mel_pad[:, 0]
